# baseline (speedup 1.0000x reference)
import functools
import math

import jax
import jax.numpy as jnp
from jax import lax
from jax.experimental import pallas as pl
from jax.experimental.pallas import tpu as pltpu
from jax.experimental.pallas import tpu_sc as plsc

_N = 10000
_NP = 10240
_F = 128
_E = 320000
_NTILES = 32
_CHUNK = 128
_NCHUNKS = 79
_PE = _NTILES * _NCHUNKS * _CHUNK


def _mm_kernel(a_ref, b_ref, o_ref, acc_ref, *, nk, zero_diag, nt, cast_bf16):
    kk = pl.program_id(2)

    @pl.when(kk == 0)
    def _init():
        acc_ref[...] = jnp.zeros_like(acc_ref)

    a = a_ref[...]
    b = b_ref[...]
    if cast_bf16:
        a = a.astype(jnp.bfloat16)
        b = b.astype(jnp.bfloat16)
    elif a.dtype != b.dtype:
        a = a.astype(jnp.float32)
        b = b.astype(jnp.float32)
    if nt:
        acc_ref[...] += lax.dot_general(
            a, b, (((1,), (1,)), ((), ())), preferred_element_type=jnp.float32)
    else:
        acc_ref[...] += jnp.dot(a, b, preferred_element_type=jnp.float32)

    @pl.when(kk == nk - 1)
    def _done():
        out = acc_ref[...]
        if zero_diag:
            i = pl.program_id(0)
            j = pl.program_id(1)
            rr = lax.broadcasted_iota(jnp.int32, out.shape, 0)
            cc = lax.broadcasted_iota(jnp.int32, out.shape, 1)
            out = jnp.where(jnp.logical_and(i == j, rr == cc), 0.0, out)
        o_ref[...] = out.astype(o_ref.dtype)


def _mm(a, b, *, nt=False, zero_diag=False, cast_bf16=False,
        out_dtype=jnp.float32):
    m, k = a.shape
    n = b.shape[0] if nt else b.shape[1]
    bm = 1280 if m % 1280 == 0 else m
    bk = 1280 if k % 1280 == 0 else k
    bn = 1280 if n % 1280 == 0 else n
    nk = k // bk
    grid = (m // bm, n // bn, nk)
    if nt:
        in_specs = [pl.BlockSpec((bm, bk), lambda i, j, q: (i, q)),
                    pl.BlockSpec((bn, bk), lambda i, j, q: (j, q))]
    else:
        in_specs = [pl.BlockSpec((bm, bk), lambda i, j, q: (i, q)),
                    pl.BlockSpec((bk, bn), lambda i, j, q: (q, j))]
    return pl.pallas_call(
        functools.partial(_mm_kernel, nk=nk, zero_diag=zero_diag, nt=nt,
                          cast_bf16=cast_bf16),
        grid=grid,
        in_specs=in_specs,
        out_specs=pl.BlockSpec((bm, bn), lambda i, j, q: (i, j)),
        out_shape=jax.ShapeDtypeStruct((m, n), out_dtype),
        scratch_shapes=[pltpu.VMEM((bm, bn), jnp.float32)],
        compiler_params=pltpu.CompilerParams(
            dimension_semantics=("parallel", "parallel", "arbitrary")),
    )(a, b)


def _sc_apply(u, srcs, dsts):
    mesh = plsc.VectorSubcoreMesh(core_axis_name="c", subcore_axis_name="s")

    @functools.partial(
        pl.kernel, mesh=mesh,
        out_type=jax.ShapeDtypeStruct((2, _NP, _F), jnp.float32),
        scratch_types=[
            pltpu.VMEM((_NCHUNKS, _CHUNK), jnp.int32),
            pltpu.VMEM((_NCHUNKS, _CHUNK), jnp.int32),
            pltpu.VMEM((_CHUNK, _F), jnp.float32),
            pltpu.VMEM_SHARED((_NP, _F), jnp.float32),
            pltpu.SemaphoreType.DMA,
        ],
    )
    def k(u_hbm, srcs_hbm, dsts_hbm, out_hbm, src_v, dst_v, rows_v, acc, sem):
        cid = lax.axis_index("c")
        sid = lax.axis_index("s")
        wid = sid * 2 + cid

        def _zrow(i, carry):
            rows_v[i // 8, pl.ds((i % 8) * 16, 16)] = jnp.zeros(
                (16,), jnp.float32)
            return carry
        lax.fori_loop(0, _CHUNK * 8, _zrow, 0)
        base = sid * (_NP // 16)
        for t in range(_NP // 16 // _CHUNK):
            pltpu.sync_copy(rows_v, acc.at[pl.ds(base + t * _CHUNK, _CHUNK)])
        pltpu.sync_copy(srcs_hbm.at[wid], src_v)
        pltpu.sync_copy(dsts_hbm.at[wid], dst_v)
        plsc.subcore_barrier()

        def _body(j, carry):
            pltpu.async_copy(u_hbm.at[src_v.at[j]], rows_v, sem).wait()
            pltpu.sync_copy(rows_v, acc.at[dst_v.at[j]], add=True)
            return carry
        lax.fori_loop(0, _NCHUNKS, _body, 0)

        plsc.subcore_barrier()
        for t in range(_NP // 16 // _CHUNK):
            sl = pl.ds(base + t * _CHUNK, _CHUNK)
            pltpu.sync_copy(acc.at[sl], out_hbm.at[cid, sl])

    out2 = k(u, srcs, dsts)
    return out2[0] + out2[1]


def _sc_translate(inv, dsts, srcs):
    mesh = plsc.VectorSubcoreMesh(core_axis_name="c", subcore_axis_name="s")

    @functools.partial(
        pl.kernel, mesh=mesh,
        out_type=(jax.ShapeDtypeStruct((_NTILES, _NCHUNKS, _CHUNK), jnp.int32),
                  jax.ShapeDtypeStruct((_NTILES, _NCHUNKS, _CHUNK), jnp.int32)),
        scratch_types=[
            pltpu.VMEM((_NCHUNKS, _CHUNK), jnp.int32),
            pltpu.VMEM((_NCHUNKS, _CHUNK), jnp.int32),
            pltpu.SemaphoreType.DMA,
        ],
    )
    def k(inv_hbm, dsts_hbm, srcs_hbm, og_hbm, oh_hbm, buf_v, out_v, sem):
        cid = lax.axis_index("c")
        sid = lax.axis_index("s")
        wid = sid * 2 + cid
        for idx_hbm, o_hbm in ((dsts_hbm, og_hbm), (srcs_hbm, oh_hbm)):
            pltpu.sync_copy(idx_hbm.at[wid], buf_v)

            def _body(j, carry):
                pltpu.async_copy(
                    inv_hbm.at[buf_v.at[j]], out_v.at[j], sem).wait()
                return carry
            lax.fori_loop(0, _NCHUNKS, _body, 0)
            pltpu.sync_copy(out_v, o_hbm.at[wid])

    return k(inv, dsts, srcs)


def _sc_degrees(srcs, dsts):
    mesh = plsc.VectorSubcoreMesh(core_axis_name="c", subcore_axis_name="s")

    @functools.partial(
        pl.kernel, mesh=mesh,
        out_type=(jax.ShapeDtypeStruct((2, _NP), jnp.float32),
                  jax.ShapeDtypeStruct((2, _NP), jnp.float32)),
        scratch_types=[
            pltpu.VMEM((_NCHUNKS, _CHUNK), jnp.int32),
            pltpu.VMEM((_NCHUNKS, _CHUNK), jnp.int32),
            pltpu.VMEM((_NCHUNKS, _CHUNK), jnp.float32),
            pltpu.VMEM((_CHUNK,), jnp.float32),
            pltpu.VMEM((_CHUNK,), jnp.float32),
            pltpu.VMEM_SHARED((_NP,), jnp.float32),
            pltpu.VMEM_SHARED((_NP,), jnp.float32),
        ],
    )
    def k(srcs_hbm, dsts_hbm, dout_hbm, cout_hbm, src_v, dst_v, self_v,
          ones_v, zero_v, dacc, cacc):
        cid = lax.axis_index("c")
        sid = lax.axis_index("s")
        wid = sid * 2 + cid
        pltpu.sync_copy(srcs_hbm.at[wid], src_v)
        pltpu.sync_copy(dsts_hbm.at[wid], dst_v)

        def _prep(i, carry):
            ones_v[pl.ds(i * 16, 16)] = jnp.ones((16,), jnp.float32)
            zero_v[pl.ds(i * 16, 16)] = jnp.zeros((16,), jnp.float32)
            return carry
        lax.fori_loop(0, _CHUNK // 16, _prep, 0)

        def _mask(i, carry):
            j = i // 8
            t = (i % 8) * 16
            s = src_v[j, pl.ds(t, 16)]
            d = dst_v[j, pl.ds(t, 16)]
            self_v[j, pl.ds(t, 16)] = jnp.where(s == d, 1.0, 0.0)
            return carry
        lax.fori_loop(0, _NCHUNKS * 8, _mask, 0)

        base = sid * (_NP // 16)
        for t in range(_NP // 16 // _CHUNK):
            sl = pl.ds(base + t * _CHUNK, _CHUNK)
            pltpu.sync_copy(zero_v, dacc.at[sl])
            pltpu.sync_copy(zero_v, cacc.at[sl])
        plsc.subcore_barrier()

        def _body(j, carry):
            pltpu.sync_copy(ones_v, dacc.at[dst_v.at[j]], add=True)
            pltpu.sync_copy(self_v.at[j], cacc.at[dst_v.at[j]], add=True)
            return carry
        lax.fori_loop(0, _NCHUNKS, _body, 0)

        plsc.subcore_barrier()
        for t in range(_NP // 16 // _CHUNK):
            sl = pl.ds(base + t * _CHUNK, _CHUNK)
            pltpu.sync_copy(dacc.at[sl], dout_hbm.at[cid, sl])
            pltpu.sync_copy(cacc.at[sl], cout_hbm.at[cid, sl])

    return k(srcs, dsts)


def _fix_kernel(g_ref, p_ref, o_ref, *, k):
    i = pl.program_id(0)
    bm, n = o_ref.shape
    g = g_ref[...]
    rows = i * bm + lax.broadcasted_iota(jnp.int32, (bm, n), 0)
    cols = lax.broadcasted_iota(jnp.int32, (bm, n), 1)
    p = p_ref[...]
    out = jnp.where(cols == p, 1.0, g)
    out = jnp.where(rows < k, out, 0.0)
    o_ref[...] = out.astype(o_ref.dtype)


def _fix(g, idxp, k, out_dtype):
    m, n = g.shape
    bm = 256
    return pl.pallas_call(
        functools.partial(_fix_kernel, k=k),
        grid=(m // bm,),
        in_specs=[pl.BlockSpec((bm, n), lambda i: (i, 0)),
                  pl.BlockSpec((bm, 1), lambda i: (i, 0))],
        out_specs=pl.BlockSpec((bm, n), lambda i: (i, 0)),
        out_shape=jax.ShapeDtypeStruct((m, n), out_dtype),
    )(g, idxp.reshape(m, 1))


def _dinv(deg):
    return jnp.where(deg > 0.0, 1.0 / jnp.sqrt(jnp.maximum(deg, 1e-12)), 0.0)


def _gcn_pooled(h, A, W, b):
    deg = jnp.sum(A, axis=1, dtype=jnp.float32) + 2.0
    di = _dinv(deg)
    u = di[:, None] * _mm(h, W)
    Av = _mm(A, u)
    return di[:, None] * (Av + 2.0 * u) + b


def _pool_scores(h, pw, n_real):
    s = jnp.tanh(jnp.dot(h, pw) / jnp.linalg.norm(pw))
    return jnp.where(jnp.arange(h.shape[0]) < n_real, s, -2.0)


def _pad_gather(h, perm, vals, kpad):
    k = perm.shape[0]
    idxp = jnp.zeros((kpad,), jnp.int32).at[:k].set(perm)
    valsp = jnp.zeros((kpad,), jnp.float32).at[:k].set(vals)
    return h[idxp] * valsp[:, None]


def _sel_rows(A, At, perm, kpad, dtype):
    k = perm.shape[0]
    idxp = jnp.zeros((kpad,), jnp.int32).at[:k].set(perm)
    G = _fix(A[idxp, :], idxp, k, dtype)
    H = _fix(At[idxp, :], idxp, k, dtype)
    return G, H


def kernel(x, edge_index, down_W0, down_b0, down_W1, down_b1, down_W2,
           down_b2, down_W3, down_b3, up_W0, up_b0, up_W1, up_b1, up_W2,
           up_b2, pool_w0, pool_w1, pool_w2):
    src = edge_index[0].astype(jnp.int32)
    dst = edge_index[1].astype(jnp.int32)

    srcp = jnp.full((_PE,), _NP - 1, jnp.int32).at[:_E].set(src)
    dstp = jnp.full((_PE,), _NP - 1, jnp.int32).at[:_E].set(dst)
    srcs = srcp.reshape(_NTILES, _NCHUNKS, _CHUNK)
    dsts = dstp.reshape(_NTILES, _NCHUNKS, _CHUNK)

    dparts, cparts = _sc_degrees(srcs, dsts)
    deg_in = dparts[0] + dparts[1]
    c0 = cparts[0] + cparts[1]
    s0 = jnp.where(c0 == 0.0, 2.0, 0.0)
    di0 = _dinv(deg_in + s0)

    xp = jnp.zeros((_NP, _F), jnp.float32).at[:_N].set(x)

    u = di0[:, None] * _mm(xp, down_W0)
    h0 = di0[:, None] * (_sc_apply(u, srcs, dsts) + s0[:, None] * u) + down_b0
    h0 = jax.nn.relu(h0)

    k1, k1p = 5000, 5120
    sc1 = _pool_scores(h0, pool_w0, _N)
    vals1, perm1 = jax.lax.top_k(sc1, k1)
    hp1 = _pad_gather(h0, perm1, vals1, k1p)
    inv1 = jnp.full((_NP,), k1p, jnp.int32).at[perm1].set(
        jnp.arange(k1, dtype=jnp.int32))
    rgp, rhp = _sc_translate(inv1, dsts, srcs)
    selfe = srcp == dstp
    rg = jnp.where(selfe, k1p, rgp.reshape(_PE))
    rh = jnp.where(selfe, k1p, rhp.reshape(_PE))
    r1 = jnp.arange(k1)
    rows_g = jnp.concatenate([rg, r1])
    cols_g = jnp.concatenate([srcp, perm1])
    rows_h = jnp.concatenate([rh, r1])
    cols_h = jnp.concatenate([dstp, perm1])
    G1 = jnp.zeros((k1p, _NP), jnp.float32).at[rows_g, cols_g].add(
        1.0).astype(jnp.bfloat16)
    H1 = jnp.zeros((k1p, _NP), jnp.float32).at[rows_h, cols_h].add(
        1.0).astype(jnp.bfloat16)
    A1 = _mm(G1, H1, nt=True, zero_diag=True)
    h1 = jax.nn.relu(_gcn_pooled(hp1, A1, down_W1, down_b1))

    k2, k2p = 2500, 2560
    A1t = A1.T
    sc2 = _pool_scores(h1, pool_w1, k1)
    vals2, perm2 = jax.lax.top_k(sc2, k2)
    hp2 = _pad_gather(h1, perm2, vals2, k2p)
    G2, H2 = _sel_rows(A1, A1t, perm2, k2p, jnp.bfloat16)
    A2 = _mm(G2, H2, nt=True, zero_diag=True)
    h2 = jax.nn.relu(_gcn_pooled(hp2, A2, down_W2, down_b2))

    k3, k3p = 1250, 1280
    A2t = A2.T
    sc3 = _pool_scores(h2, pool_w2, k2)
    vals3, perm3 = jax.lax.top_k(sc3, k3)
    hp3 = _pad_gather(h2, perm3, vals3, k3p)
    G3, H3 = _sel_rows(A2, A2t, perm3, k3p, jnp.float32)
    A3 = _mm(G3, H3, nt=True, zero_diag=True)
    h3 = jax.nn.relu(_gcn_pooled(hp3, A3, down_W3, down_b3))

    u0 = h2 + jnp.zeros_like(h2).at[perm3].set(h3[:k3])
    g = jax.nn.relu(_gcn_pooled(u0, A2, up_W0, up_b0))

    u1 = h1 + jnp.zeros_like(h1).at[perm2].set(g[:k2])
    g = jax.nn.relu(_gcn_pooled(u1, A1, up_W1, up_b1))

    u2 = h0 + jnp.zeros_like(h0).at[perm1].set(g[:k1])
    v = di0[:, None] * _mm(u2, up_W2)
    out = di0[:, None] * (_sc_apply(v, srcs, dsts) + s0[:, None] * v) + up_b2

    return out[:_N]


# speedup vs baseline: 1.8868x; 1.8868x over previous
import functools
import math

import jax
import jax.numpy as jnp
from jax import lax
from jax.experimental import pallas as pl
from jax.experimental.pallas import tpu as pltpu
from jax.experimental.pallas import tpu_sc as plsc

_N = 10000
_NP = 10240
_F = 128
_E = 320000
_NTILES = 32
_CHUNK = 128
_NCHUNKS = 79
_PE = _NTILES * _NCHUNKS * _CHUNK


def _mm_kernel(a_ref, b_ref, o_ref, acc_ref, *, nk, zero_diag, nt, cast_bf16):
    kk = pl.program_id(2)

    @pl.when(kk == 0)
    def _init():
        acc_ref[...] = jnp.zeros_like(acc_ref)

    a = a_ref[...]
    b = b_ref[...]
    if cast_bf16:
        a = a.astype(jnp.bfloat16)
        b = b.astype(jnp.bfloat16)
    elif a.dtype != b.dtype:
        a = a.astype(jnp.float32)
        b = b.astype(jnp.float32)
    if nt:
        acc_ref[...] += lax.dot_general(
            a, b, (((1,), (1,)), ((), ())), preferred_element_type=jnp.float32)
    else:
        acc_ref[...] += jnp.dot(a, b, preferred_element_type=jnp.float32)

    @pl.when(kk == nk - 1)
    def _done():
        out = acc_ref[...]
        if zero_diag:
            i = pl.program_id(0)
            j = pl.program_id(1)
            rr = lax.broadcasted_iota(jnp.int32, out.shape, 0)
            cc = lax.broadcasted_iota(jnp.int32, out.shape, 1)
            out = jnp.where(jnp.logical_and(i == j, rr == cc), 0.0, out)
        o_ref[...] = out.astype(o_ref.dtype)


def _mm(a, b, *, nt=False, zero_diag=False, cast_bf16=False,
        out_dtype=jnp.float32):
    m, k = a.shape
    n = b.shape[0] if nt else b.shape[1]
    bm = 1280 if m % 1280 == 0 else m
    bk = 1280 if k % 1280 == 0 else k
    bn = 1280 if n % 1280 == 0 else n
    nk = k // bk
    grid = (m // bm, n // bn, nk)
    if nt:
        in_specs = [pl.BlockSpec((bm, bk), lambda i, j, q: (i, q)),
                    pl.BlockSpec((bn, bk), lambda i, j, q: (j, q))]
    else:
        in_specs = [pl.BlockSpec((bm, bk), lambda i, j, q: (i, q)),
                    pl.BlockSpec((bk, bn), lambda i, j, q: (q, j))]
    return pl.pallas_call(
        functools.partial(_mm_kernel, nk=nk, zero_diag=zero_diag, nt=nt,
                          cast_bf16=cast_bf16),
        grid=grid,
        in_specs=in_specs,
        out_specs=pl.BlockSpec((bm, bn), lambda i, j, q: (i, j)),
        out_shape=jax.ShapeDtypeStruct((m, n), out_dtype),
        scratch_shapes=[pltpu.VMEM((bm, bn), jnp.float32)],
        compiler_params=pltpu.CompilerParams(
            dimension_semantics=("parallel", "parallel", "arbitrary")),
    )(a, b)


def _sc_apply(u, srcs, dsts):
    mesh = plsc.VectorSubcoreMesh(core_axis_name="c", subcore_axis_name="s")

    @functools.partial(
        pl.kernel, mesh=mesh,
        out_type=jax.ShapeDtypeStruct((2, _NP, _F), jnp.float32),
        scratch_types=[
            pltpu.VMEM((_NCHUNKS, _CHUNK), jnp.int32),
            pltpu.VMEM((_NCHUNKS, _CHUNK), jnp.int32),
            pltpu.VMEM((_CHUNK, _F), jnp.float32),
            pltpu.VMEM_SHARED((_NP, _F), jnp.float32),
            pltpu.SemaphoreType.DMA,
        ],
    )
    def k(u_hbm, srcs_hbm, dsts_hbm, out_hbm, src_v, dst_v, rows_v, acc, sem):
        cid = lax.axis_index("c")
        sid = lax.axis_index("s")
        wid = sid * 2 + cid

        def _zrow(i, carry):
            rows_v[i // 8, pl.ds((i % 8) * 16, 16)] = jnp.zeros(
                (16,), jnp.float32)
            return carry
        lax.fori_loop(0, _CHUNK * 8, _zrow, 0)
        base = sid * (_NP // 16)
        for t in range(_NP // 16 // _CHUNK):
            pltpu.sync_copy(rows_v, acc.at[pl.ds(base + t * _CHUNK, _CHUNK)])
        pltpu.sync_copy(srcs_hbm.at[wid], src_v)
        pltpu.sync_copy(dsts_hbm.at[wid], dst_v)
        plsc.subcore_barrier()

        def _body(j, carry):
            pltpu.async_copy(u_hbm.at[src_v.at[j]], rows_v, sem).wait()
            pltpu.sync_copy(rows_v, acc.at[dst_v.at[j]], add=True)
            return carry
        lax.fori_loop(0, _NCHUNKS, _body, 0)

        plsc.subcore_barrier()
        for t in range(_NP // 16 // _CHUNK):
            sl = pl.ds(base + t * _CHUNK, _CHUNK)
            pltpu.sync_copy(acc.at[sl], out_hbm.at[cid, sl])

    out2 = k(u, srcs, dsts)
    return out2[0] + out2[1]


def _sc_translate(inv, dsts, srcs):
    mesh = plsc.VectorSubcoreMesh(core_axis_name="c", subcore_axis_name="s")

    @functools.partial(
        pl.kernel, mesh=mesh,
        out_type=(jax.ShapeDtypeStruct((_NTILES, _NCHUNKS, _CHUNK), jnp.int32),
                  jax.ShapeDtypeStruct((_NTILES, _NCHUNKS, _CHUNK), jnp.int32)),
        scratch_types=[
            pltpu.VMEM((_NCHUNKS, _CHUNK), jnp.int32),
            pltpu.VMEM((_NCHUNKS, _CHUNK), jnp.int32),
            pltpu.SemaphoreType.DMA,
        ],
    )
    def k(inv_hbm, dsts_hbm, srcs_hbm, og_hbm, oh_hbm, buf_v, out_v, sem):
        cid = lax.axis_index("c")
        sid = lax.axis_index("s")
        wid = sid * 2 + cid
        for idx_hbm, o_hbm in ((dsts_hbm, og_hbm), (srcs_hbm, oh_hbm)):
            pltpu.sync_copy(idx_hbm.at[wid], buf_v)

            def _body(j, carry):
                pltpu.async_copy(
                    inv_hbm.at[buf_v.at[j]], out_v.at[j], sem).wait()
                return carry
            lax.fori_loop(0, _NCHUNKS, _body, 0)
            pltpu.sync_copy(out_v, o_hbm.at[wid])

    return k(inv, dsts, srcs)


def _sc_degrees(srcs, dsts):
    mesh = plsc.VectorSubcoreMesh(core_axis_name="c", subcore_axis_name="s")

    @functools.partial(
        pl.kernel, mesh=mesh,
        out_type=(jax.ShapeDtypeStruct((2, _NP), jnp.float32),
                  jax.ShapeDtypeStruct((2, _NP), jnp.float32)),
        scratch_types=[
            pltpu.VMEM((_NCHUNKS, _CHUNK), jnp.int32),
            pltpu.VMEM((_NCHUNKS, _CHUNK), jnp.int32),
            pltpu.VMEM((_NCHUNKS, _CHUNK), jnp.float32),
            pltpu.VMEM((_CHUNK,), jnp.float32),
            pltpu.VMEM((_CHUNK,), jnp.float32),
            pltpu.VMEM_SHARED((_NP,), jnp.float32),
            pltpu.VMEM_SHARED((_NP,), jnp.float32),
        ],
    )
    def k(srcs_hbm, dsts_hbm, dout_hbm, cout_hbm, src_v, dst_v, self_v,
          ones_v, zero_v, dacc, cacc):
        cid = lax.axis_index("c")
        sid = lax.axis_index("s")
        wid = sid * 2 + cid
        pltpu.sync_copy(srcs_hbm.at[wid], src_v)
        pltpu.sync_copy(dsts_hbm.at[wid], dst_v)

        def _prep(i, carry):
            ones_v[pl.ds(i * 16, 16)] = jnp.ones((16,), jnp.float32)
            zero_v[pl.ds(i * 16, 16)] = jnp.zeros((16,), jnp.float32)
            return carry
        lax.fori_loop(0, _CHUNK // 16, _prep, 0)

        def _mask(i, carry):
            j = i // 8
            t = (i % 8) * 16
            s = src_v[j, pl.ds(t, 16)]
            d = dst_v[j, pl.ds(t, 16)]
            self_v[j, pl.ds(t, 16)] = jnp.where(s == d, 1.0, 0.0)
            return carry
        lax.fori_loop(0, _NCHUNKS * 8, _mask, 0)

        base = sid * (_NP // 16)
        for t in range(_NP // 16 // _CHUNK):
            sl = pl.ds(base + t * _CHUNK, _CHUNK)
            pltpu.sync_copy(zero_v, dacc.at[sl])
            pltpu.sync_copy(zero_v, cacc.at[sl])
        plsc.subcore_barrier()

        def _body(j, carry):
            pltpu.sync_copy(ones_v, dacc.at[dst_v.at[j]], add=True)
            pltpu.sync_copy(self_v.at[j], cacc.at[dst_v.at[j]], add=True)
            return carry
        lax.fori_loop(0, _NCHUNKS, _body, 0)

        plsc.subcore_barrier()
        for t in range(_NP // 16 // _CHUNK):
            sl = pl.ds(base + t * _CHUNK, _CHUNK)
            pltpu.sync_copy(dacc.at[sl], dout_hbm.at[cid, sl])
            pltpu.sync_copy(cacc.at[sl], cout_hbm.at[cid, sl])

    return k(srcs, dsts)


def _fix_kernel(g_ref, p_ref, o_ref, *, k):
    i = pl.program_id(0)
    bm, n = o_ref.shape
    g = g_ref[...]
    rows = i * bm + lax.broadcasted_iota(jnp.int32, (bm, n), 0)
    cols = lax.broadcasted_iota(jnp.int32, (bm, n), 1)
    p = p_ref[...]
    out = jnp.where(cols == p, 1.0, g)
    out = jnp.where(rows < k, out, 0.0)
    o_ref[...] = out.astype(o_ref.dtype)


def _fix(g, idxp, k, out_dtype):
    m, n = g.shape
    bm = 256
    return pl.pallas_call(
        functools.partial(_fix_kernel, k=k),
        grid=(m // bm,),
        in_specs=[pl.BlockSpec((bm, n), lambda i: (i, 0)),
                  pl.BlockSpec((bm, 1), lambda i: (i, 0))],
        out_specs=pl.BlockSpec((bm, n), lambda i: (i, 0)),
        out_shape=jax.ShapeDtypeStruct((m, n), out_dtype),
    )(g, idxp.reshape(m, 1))


def _dinv(deg):
    return jnp.where(deg > 0.0, 1.0 / jnp.sqrt(jnp.maximum(deg, 1e-12)), 0.0)


def _gcn_pooled(h, A, W, b):
    deg = jnp.sum(A, axis=1, dtype=jnp.float32) + 2.0
    di = _dinv(deg)
    u = di[:, None] * _mm(h, W)
    Av = _mm(A, u)
    return di[:, None] * (Av + 2.0 * u) + b


def _pool_scores(h, pw, n_real):
    s = jnp.tanh(jnp.dot(h, pw) / jnp.linalg.norm(pw))
    return jnp.where(jnp.arange(h.shape[0]) < n_real, s, -2.0)


def _pad_gather(h, perm, vals, kpad):
    k = perm.shape[0]
    idxp = jnp.zeros((kpad,), jnp.int32).at[:k].set(perm)
    valsp = jnp.zeros((kpad,), jnp.float32).at[:k].set(vals)
    return h[idxp] * valsp[:, None]


def _sel_rows(A, At, perm, kpad, dtype):
    k = perm.shape[0]
    idxp = jnp.zeros((kpad,), jnp.int32).at[:k].set(perm)
    G = _fix(A[idxp, :], idxp, k, dtype)
    H = _fix(At[idxp, :], idxp, k, dtype)
    return G, H


def kernel(x, edge_index, down_W0, down_b0, down_W1, down_b1, down_W2,
           down_b2, down_W3, down_b3, up_W0, up_b0, up_W1, up_b1, up_W2,
           up_b2, pool_w0, pool_w1, pool_w2):
    src = edge_index[0].astype(jnp.int32)
    dst = edge_index[1].astype(jnp.int32)

    srcp = jnp.full((_PE,), _NP - 1, jnp.int32).at[:_E].set(src)
    dstp = jnp.full((_PE,), _NP - 1, jnp.int32).at[:_E].set(dst)
    srcs = srcp.reshape(_NTILES, _NCHUNKS, _CHUNK)
    dsts = dstp.reshape(_NTILES, _NCHUNKS, _CHUNK)

    dparts, cparts = _sc_degrees(srcs, dsts)
    deg_in = dparts[0] + dparts[1]
    c0 = cparts[0] + cparts[1]
    s0 = jnp.where(c0 == 0.0, 2.0, 0.0)
    di0 = _dinv(deg_in + s0)

    xp = jnp.zeros((_NP, _F), jnp.float32).at[:_N].set(x)

    u = di0[:, None] * _mm(xp, down_W0)
    h0 = di0[:, None] * (_sc_apply(u, srcs, dsts) + s0[:, None] * u) + down_b0
    h0 = jax.nn.relu(h0)

    k1, k1p = 5000, 5120
    sc1 = _pool_scores(h0, pool_w0, _N)
    vals1, perm1 = jax.lax.top_k(sc1, k1)
    hp1 = _pad_gather(h0, perm1, vals1, k1p)
    inv1 = jnp.full((_NP,), k1p, jnp.int32).at[perm1].set(
        jnp.arange(k1, dtype=jnp.int32))
    rgp, rhp = _sc_translate(inv1, dsts, srcs)
    idx1p = jnp.zeros((k1p,), jnp.int32).at[:k1].set(perm1)
    rg = rgp.reshape(_PE)
    rh = rhp.reshape(_PE)
    G1r = jnp.zeros((k1p, _NP), jnp.float32).at[rg, srcp].add(1.0)
    H1r = jnp.zeros((k1p, _NP), jnp.float32).at[rh, dstp].add(1.0)
    G1 = _fix(G1r, idx1p, k1, jnp.bfloat16)
    H1 = _fix(H1r, idx1p, k1, jnp.bfloat16)
    A1 = _mm(G1, H1, nt=True, zero_diag=True)
    h1 = jax.nn.relu(_gcn_pooled(hp1, A1, down_W1, down_b1))

    k2, k2p = 2500, 2560
    A1t = A1.T
    sc2 = _pool_scores(h1, pool_w1, k1)
    vals2, perm2 = jax.lax.top_k(sc2, k2)
    hp2 = _pad_gather(h1, perm2, vals2, k2p)
    G2, H2 = _sel_rows(A1, A1t, perm2, k2p, jnp.bfloat16)
    A2 = _mm(G2, H2, nt=True, zero_diag=True)
    h2 = jax.nn.relu(_gcn_pooled(hp2, A2, down_W2, down_b2))

    k3, k3p = 1250, 1280
    A2t = A2.T
    sc3 = _pool_scores(h2, pool_w2, k2)
    vals3, perm3 = jax.lax.top_k(sc3, k3)
    hp3 = _pad_gather(h2, perm3, vals3, k3p)
    G3, H3 = _sel_rows(A2, A2t, perm3, k3p, jnp.float32)
    A3 = _mm(G3, H3, nt=True, zero_diag=True)
    h3 = jax.nn.relu(_gcn_pooled(hp3, A3, down_W3, down_b3))

    u0 = h2 + jnp.zeros_like(h2).at[perm3].set(h3[:k3])
    g = jax.nn.relu(_gcn_pooled(u0, A2, up_W0, up_b0))

    u1 = h1 + jnp.zeros_like(h1).at[perm2].set(g[:k2])
    g = jax.nn.relu(_gcn_pooled(u1, A1, up_W1, up_b1))

    u2 = h0 + jnp.zeros_like(h0).at[perm1].set(g[:k1])
    v = di0[:, None] * _mm(u2, up_W2)
    out = di0[:, None] * (_sc_apply(v, srcs, dsts) + s0[:, None] * v) + up_b2

    return out[:_N]


# speedup vs baseline: 1.8886x; 1.0010x over previous
import functools
import math

import jax
import jax.numpy as jnp
from jax import lax
from jax.experimental import pallas as pl
from jax.experimental.pallas import tpu as pltpu
from jax.experimental.pallas import tpu_sc as plsc

_N = 10000
_NP = 10240
_F = 128
_E = 320000
_NTILES = 32
_CHUNK = 128
_NCHUNKS = 79
_PE = _NTILES * _NCHUNKS * _CHUNK


def _mm_kernel(a_ref, b_ref, o_ref, acc_ref, *, nk, zero_diag, nt, cast_bf16):
    kk = pl.program_id(2)

    @pl.when(kk == 0)
    def _init():
        acc_ref[...] = jnp.zeros_like(acc_ref)

    a = a_ref[...]
    b = b_ref[...]
    if cast_bf16:
        a = a.astype(jnp.bfloat16)
        b = b.astype(jnp.bfloat16)
    elif a.dtype != b.dtype:
        a = a.astype(jnp.float32)
        b = b.astype(jnp.float32)
    if nt:
        acc_ref[...] += lax.dot_general(
            a, b, (((1,), (1,)), ((), ())), preferred_element_type=jnp.float32)
    else:
        acc_ref[...] += jnp.dot(a, b, preferred_element_type=jnp.float32)

    @pl.when(kk == nk - 1)
    def _done():
        out = acc_ref[...]
        if zero_diag:
            i = pl.program_id(0)
            j = pl.program_id(1)
            rr = lax.broadcasted_iota(jnp.int32, out.shape, 0)
            cc = lax.broadcasted_iota(jnp.int32, out.shape, 1)
            out = jnp.where(jnp.logical_and(i == j, rr == cc), 0.0, out)
        o_ref[...] = out.astype(o_ref.dtype)


def _mm(a, b, *, nt=False, zero_diag=False, cast_bf16=False,
        out_dtype=jnp.float32):
    m, k = a.shape
    n = b.shape[0] if nt else b.shape[1]
    bm = 1280 if m % 1280 == 0 else m
    bk = 1280 if k % 1280 == 0 else k
    bn = 1280 if n % 1280 == 0 else n
    nk = k // bk
    grid = (m // bm, n // bn, nk)
    if nt:
        in_specs = [pl.BlockSpec((bm, bk), lambda i, j, q: (i, q)),
                    pl.BlockSpec((bn, bk), lambda i, j, q: (j, q))]
    else:
        in_specs = [pl.BlockSpec((bm, bk), lambda i, j, q: (i, q)),
                    pl.BlockSpec((bk, bn), lambda i, j, q: (q, j))]
    return pl.pallas_call(
        functools.partial(_mm_kernel, nk=nk, zero_diag=zero_diag, nt=nt,
                          cast_bf16=cast_bf16),
        grid=grid,
        in_specs=in_specs,
        out_specs=pl.BlockSpec((bm, bn), lambda i, j, q: (i, j)),
        out_shape=jax.ShapeDtypeStruct((m, n), out_dtype),
        scratch_shapes=[pltpu.VMEM((bm, bn), jnp.float32)],
        compiler_params=pltpu.CompilerParams(
            dimension_semantics=("parallel", "parallel", "arbitrary")),
    )(a, b)


def _sc_apply(u, srcs, dsts):
    mesh = plsc.VectorSubcoreMesh(core_axis_name="c", subcore_axis_name="s")

    @functools.partial(
        pl.kernel, mesh=mesh,
        out_type=jax.ShapeDtypeStruct((2, _NP, _F), jnp.float32),
        scratch_types=[
            pltpu.VMEM((_NCHUNKS, _CHUNK), jnp.int32),
            pltpu.VMEM((_NCHUNKS, _CHUNK), jnp.int32),
            pltpu.VMEM((_CHUNK, _F), jnp.float32),
            pltpu.VMEM_SHARED((_NP, _F), jnp.float32),
            pltpu.SemaphoreType.DMA,
        ],
    )
    def k(u_hbm, srcs_hbm, dsts_hbm, out_hbm, src_v, dst_v, rows_a,
          acc, sem_a):
        cid = lax.axis_index("c")
        sid = lax.axis_index("s")
        wid = sid * 2 + cid

        def _zrow(i, carry):
            rows_a[i // 8, pl.ds((i % 8) * 16, 16)] = jnp.zeros(
                (16,), jnp.float32)
            return carry
        lax.fori_loop(0, _CHUNK * 8, _zrow, 0)
        base = sid * (_NP // 16)
        for t in range(_NP // 16 // _CHUNK):
            pltpu.sync_copy(rows_a, acc.at[pl.ds(base + t * _CHUNK, _CHUNK)])
        pltpu.sync_copy(srcs_hbm.at[wid], src_v)
        pltpu.sync_copy(dsts_hbm.at[wid], dst_v)
        plsc.subcore_barrier()

        def _body(j, carry):
            pltpu.async_copy(u_hbm.at[src_v.at[j]], rows_a, sem_a).wait()
            pltpu.sync_copy(rows_a, acc.at[dst_v.at[j]], add=True)
            return carry
        lax.fori_loop(0, _NCHUNKS, _body, 0)

        plsc.subcore_barrier()
        for t in range(_NP // 16 // _CHUNK):
            sl = pl.ds(base + t * _CHUNK, _CHUNK)
            pltpu.sync_copy(acc.at[sl], out_hbm.at[cid, sl])

    out2 = k(u, srcs, dsts)
    return out2[0] + out2[1]


def _sc_translate(inv, dsts, srcs):
    mesh = plsc.VectorSubcoreMesh(core_axis_name="c", subcore_axis_name="s")

    @functools.partial(
        pl.kernel, mesh=mesh,
        out_type=(jax.ShapeDtypeStruct((_NTILES, _NCHUNKS, _CHUNK), jnp.int32),
                  jax.ShapeDtypeStruct((_NTILES, _NCHUNKS, _CHUNK), jnp.int32)),
        scratch_types=[
            pltpu.VMEM((_NCHUNKS, _CHUNK), jnp.int32),
            pltpu.VMEM((_NCHUNKS, _CHUNK), jnp.int32),
            pltpu.SemaphoreType.DMA,
        ],
    )
    def k(inv_hbm, dsts_hbm, srcs_hbm, og_hbm, oh_hbm, buf_v, out_v, sem):
        cid = lax.axis_index("c")
        sid = lax.axis_index("s")
        wid = sid * 2 + cid
        for idx_hbm, o_hbm in ((dsts_hbm, og_hbm), (srcs_hbm, oh_hbm)):
            pltpu.sync_copy(idx_hbm.at[wid], buf_v)

            def _body(j, carry):
                pltpu.async_copy(
                    inv_hbm.at[buf_v.at[j]], out_v.at[j], sem).wait()
                return carry
            lax.fori_loop(0, _NCHUNKS, _body, 0)
            pltpu.sync_copy(out_v, o_hbm.at[wid])

    return k(inv, dsts, srcs)


def _sc_degrees(srcs, dsts):
    mesh = plsc.VectorSubcoreMesh(core_axis_name="c", subcore_axis_name="s")

    @functools.partial(
        pl.kernel, mesh=mesh,
        out_type=(jax.ShapeDtypeStruct((2, _NP), jnp.float32),
                  jax.ShapeDtypeStruct((2, _NP), jnp.float32)),
        scratch_types=[
            pltpu.VMEM((_NCHUNKS, _CHUNK), jnp.int32),
            pltpu.VMEM((_NCHUNKS, _CHUNK), jnp.int32),
            pltpu.VMEM((_NCHUNKS, _CHUNK), jnp.float32),
            pltpu.VMEM((_CHUNK,), jnp.float32),
            pltpu.VMEM((_CHUNK,), jnp.float32),
            pltpu.VMEM_SHARED((_NP,), jnp.float32),
            pltpu.VMEM_SHARED((_NP,), jnp.float32),
        ],
    )
    def k(srcs_hbm, dsts_hbm, dout_hbm, cout_hbm, src_v, dst_v, self_v,
          ones_v, zero_v, dacc, cacc):
        cid = lax.axis_index("c")
        sid = lax.axis_index("s")
        wid = sid * 2 + cid
        pltpu.sync_copy(srcs_hbm.at[wid], src_v)
        pltpu.sync_copy(dsts_hbm.at[wid], dst_v)

        def _prep(i, carry):
            ones_v[pl.ds(i * 16, 16)] = jnp.ones((16,), jnp.float32)
            zero_v[pl.ds(i * 16, 16)] = jnp.zeros((16,), jnp.float32)
            return carry
        lax.fori_loop(0, _CHUNK // 16, _prep, 0)

        def _mask(i, carry):
            j = i // 8
            t = (i % 8) * 16
            s = src_v[j, pl.ds(t, 16)]
            d = dst_v[j, pl.ds(t, 16)]
            self_v[j, pl.ds(t, 16)] = jnp.where(s == d, 1.0, 0.0)
            return carry
        lax.fori_loop(0, _NCHUNKS * 8, _mask, 0)

        base = sid * (_NP // 16)
        for t in range(_NP // 16 // _CHUNK):
            sl = pl.ds(base + t * _CHUNK, _CHUNK)
            pltpu.sync_copy(zero_v, dacc.at[sl])
            pltpu.sync_copy(zero_v, cacc.at[sl])
        plsc.subcore_barrier()

        def _body(j, carry):
            pltpu.sync_copy(ones_v, dacc.at[dst_v.at[j]], add=True)
            pltpu.sync_copy(self_v.at[j], cacc.at[dst_v.at[j]], add=True)
            return carry
        lax.fori_loop(0, _NCHUNKS, _body, 0)

        plsc.subcore_barrier()
        for t in range(_NP // 16 // _CHUNK):
            sl = pl.ds(base + t * _CHUNK, _CHUNK)
            pltpu.sync_copy(dacc.at[sl], dout_hbm.at[cid, sl])
            pltpu.sync_copy(cacc.at[sl], cout_hbm.at[cid, sl])

    return k(srcs, dsts)


def _fix_kernel(g_ref, p_ref, o_ref, *, k):
    i = pl.program_id(0)
    bm, n = o_ref.shape
    g = g_ref[...]
    rows = i * bm + lax.broadcasted_iota(jnp.int32, (bm, n), 0)
    cols = lax.broadcasted_iota(jnp.int32, (bm, n), 1)
    p = p_ref[...]
    out = jnp.where(cols == p, 1.0, g)
    out = jnp.where(rows < k, out, 0.0)
    o_ref[...] = out.astype(o_ref.dtype)


def _fix(g, idxp, k, out_dtype):
    m, n = g.shape
    bm = 256
    return pl.pallas_call(
        functools.partial(_fix_kernel, k=k),
        grid=(m // bm,),
        in_specs=[pl.BlockSpec((bm, n), lambda i: (i, 0)),
                  pl.BlockSpec((bm, 1), lambda i: (i, 0))],
        out_specs=pl.BlockSpec((bm, n), lambda i: (i, 0)),
        out_shape=jax.ShapeDtypeStruct((m, n), out_dtype),
    )(g, idxp.reshape(m, 1))


def _dinv(deg):
    return jnp.where(deg > 0.0, 1.0 / jnp.sqrt(jnp.maximum(deg, 1e-12)), 0.0)


def _gcn_pooled(h, A, W, b):
    deg = jnp.sum(A, axis=1, dtype=jnp.float32) + 2.0
    di = _dinv(deg)
    u = di[:, None] * _mm(h, W)
    Av = _mm(A, u)
    return di[:, None] * (Av + 2.0 * u) + b


def _pool_scores(h, pw, n_real):
    s = jnp.tanh(jnp.dot(h, pw) / jnp.linalg.norm(pw))
    return jnp.where(jnp.arange(h.shape[0]) < n_real, s, -2.0)


def _pad_gather(h, perm, vals, kpad):
    k = perm.shape[0]
    idxp = jnp.zeros((kpad,), jnp.int32).at[:k].set(perm)
    valsp = jnp.zeros((kpad,), jnp.float32).at[:k].set(vals)
    return h[idxp] * valsp[:, None]


def _sel_rows(A, At, perm, kpad, dtype):
    k = perm.shape[0]
    idxp = jnp.zeros((kpad,), jnp.int32).at[:k].set(perm)
    G = _fix(A[idxp, :], idxp, k, dtype)
    H = _fix(At[idxp, :], idxp, k, dtype)
    return G, H


def kernel(x, edge_index, down_W0, down_b0, down_W1, down_b1, down_W2,
           down_b2, down_W3, down_b3, up_W0, up_b0, up_W1, up_b1, up_W2,
           up_b2, pool_w0, pool_w1, pool_w2):
    src = edge_index[0].astype(jnp.int32)
    dst = edge_index[1].astype(jnp.int32)

    srcp = jnp.full((_PE,), _NP - 1, jnp.int32).at[:_E].set(src)
    dstp = jnp.full((_PE,), _NP - 1, jnp.int32).at[:_E].set(dst)
    srcs = srcp.reshape(_NTILES, _NCHUNKS, _CHUNK)
    dsts = dstp.reshape(_NTILES, _NCHUNKS, _CHUNK)

    dparts, cparts = _sc_degrees(srcs, dsts)
    deg_in = dparts[0] + dparts[1]
    c0 = cparts[0] + cparts[1]
    s0 = jnp.where(c0 == 0.0, 2.0, 0.0)
    di0 = _dinv(deg_in + s0)

    xp = jnp.zeros((_NP, _F), jnp.float32).at[:_N].set(x)

    u = di0[:, None] * _mm(xp, down_W0)
    h0 = di0[:, None] * (_sc_apply(u, srcs, dsts) + s0[:, None] * u) + down_b0
    h0 = jax.nn.relu(h0)

    k1, k1p = 5000, 5120
    sc1 = _pool_scores(h0, pool_w0, _N)
    vals1, perm1 = jax.lax.top_k(sc1, k1)
    hp1 = _pad_gather(h0, perm1, vals1, k1p)
    inv1 = jnp.full((_NP,), k1p, jnp.int32).at[perm1].set(
        jnp.arange(k1, dtype=jnp.int32))
    rgp, rhp = _sc_translate(inv1, dsts, srcs)
    idx1p = jnp.zeros((k1p,), jnp.int32).at[:k1].set(perm1)
    rg = rgp.reshape(_PE)
    rh = rhp.reshape(_PE)
    G1r = jnp.zeros((k1p, _NP), jnp.float32).at[rg, srcp].add(1.0)
    H1r = jnp.zeros((k1p, _NP), jnp.float32).at[rh, dstp].add(1.0)
    G1 = _fix(G1r, idx1p, k1, jnp.bfloat16)
    H1 = _fix(H1r, idx1p, k1, jnp.bfloat16)
    A1 = _mm(G1, H1, nt=True, zero_diag=True)
    h1 = jax.nn.relu(_gcn_pooled(hp1, A1, down_W1, down_b1))

    k2, k2p = 2500, 2560
    A1t = A1.T
    sc2 = _pool_scores(h1, pool_w1, k1)
    vals2, perm2 = jax.lax.top_k(sc2, k2)
    hp2 = _pad_gather(h1, perm2, vals2, k2p)
    G2, H2 = _sel_rows(A1, A1t, perm2, k2p, jnp.bfloat16)
    A2 = _mm(G2, H2, nt=True, zero_diag=True)
    h2 = jax.nn.relu(_gcn_pooled(hp2, A2, down_W2, down_b2))

    k3, k3p = 1250, 1280
    A2t = A2.T
    sc3 = _pool_scores(h2, pool_w2, k2)
    vals3, perm3 = jax.lax.top_k(sc3, k3)
    hp3 = _pad_gather(h2, perm3, vals3, k3p)
    G3, H3 = _sel_rows(A2, A2t, perm3, k3p, jnp.float32)
    A3 = _mm(G3, H3, nt=True, zero_diag=True)
    h3 = jax.nn.relu(_gcn_pooled(hp3, A3, down_W3, down_b3))

    u0 = h2 + jnp.zeros_like(h2).at[perm3].set(h3[:k3])
    g = jax.nn.relu(_gcn_pooled(u0, A2, up_W0, up_b0))

    u1 = h1 + jnp.zeros_like(h1).at[perm2].set(g[:k2])
    g = jax.nn.relu(_gcn_pooled(u1, A1, up_W1, up_b1))

    u2 = h0 + jnp.zeros_like(h0).at[perm1].set(g[:k1])
    v = di0[:, None] * _mm(u2, up_W2)
    out = di0[:, None] * (_sc_apply(v, srcs, dsts) + s0[:, None] * v) + up_b2

    return out[:_N]
